# Initial kernel scaffold; baseline (speedup 1.0000x reference)
import jax
import jax.numpy as jnp
from jax.experimental import pallas as pl


def kernel(x, edge_index, edge_attr, batch, W0, b0, W1, b1, W2, b2, Wf, bf):
    raise NotImplementedError("write your pallas kernel here")


# speedup vs baseline: 6.8814x; 6.8814x over previous
import functools
import jax
import jax.numpy as jnp
from jax import lax
from jax.experimental import pallas as pl
from jax.experimental.pallas import tpu as pltpu
from jax.experimental.pallas import tpu_sc as plsc

_N = 10000
_E = 320000
_D = 128
_H = 128
_OUT = 16
_G = 64

_NC = 2
_NS = 16
_NW = _NC * _NS
_CHUNK = 64
_NBUF = 2
_NCHUNK = 160
_EPT = _NCHUNK * _CHUNK
_EPAD = _NW * _EPT
_NPAD = 10112
_NBLK = _NPAD // 128
_RPT = _NPAD // _NS
_DUMP = _NPAD - 1


def _deg_body(dst3_hbm, ones_hbm, zeros8_hbm, degp_hbm, dst_v, ones_v, acc8):
    c = lax.axis_index("c")
    s = lax.axis_index("s")
    tid = c * _NS + s
    pltpu.sync_copy(dst3_hbm.at[tid], dst_v)
    pltpu.sync_copy(ones_hbm, ones_v)
    pltpu.sync_copy(zeros8_hbm.at[pl.ds(s * _RPT, _RPT)],
                    acc8.at[pl.ds(s * _RPT, _RPT)])
    plsc.subcore_barrier()

    def body(j, carry):
        pltpu.sync_copy(ones_v, acc8.at[dst_v.at[j]], add=True)
        return carry

    lax.fori_loop(0, _NCHUNK, body, 0)
    plsc.subcore_barrier()
    pltpu.sync_copy(acc8.at[pl.ds(s * _RPT, _RPT)],
                    degp_hbm.at[c, pl.ds(s * _RPT, _RPT)])


@functools.cache
def _deg_kernel():
    return pl.kernel(
        _deg_body,
        out_type=jax.ShapeDtypeStruct((_NC, _NPAD, _H), jnp.float32),
        mesh=plsc.VectorSubcoreMesh(core_axis_name="c", subcore_axis_name="s",
                                    num_cores=_NC, num_subcores=_NS),
        scratch_types=[
            pltpu.VMEM((_NCHUNK, _CHUNK), jnp.int32),
            pltpu.VMEM((_CHUNK, _H), jnp.float32),
            pltpu.VMEM_SHARED((_NPAD, _H), jnp.float32),
        ],
    )


def _prop_body(src_hbm, dst3_hbm, s_hbm, zeros_hbm, out_hbm,
               src_v, dst_v, rows_v, acc, sem0, sem1):
    c = lax.axis_index("c")
    s = lax.axis_index("s")
    tid = c * _NS + s
    sems = (sem0, sem1)

    pltpu.sync_copy(src_hbm.at[pl.ds(tid * _EPT, _EPT)], src_v)
    pltpu.sync_copy(dst3_hbm.at[tid], dst_v)
    for b in range(_NBUF):
        pltpu.async_copy(s_hbm.at[src_v.at[pl.ds(b * _CHUNK, _CHUNK)]],
                         rows_v.at[b], sems[b])
    pltpu.sync_copy(zeros_hbm.at[pl.ds(s * _RPT, _RPT)],
                    acc.at[pl.ds(s * _RPT, _RPT)])
    plsc.subcore_barrier()

    def outer(jo, carry):
        for b in range(_NBUF):
            j = jo * _NBUF + b
            pltpu.make_async_copy(
                s_hbm.at[src_v.at[pl.ds(j * _CHUNK, _CHUNK)]],
                rows_v.at[b], sems[b]).wait()
            pltpu.sync_copy(rows_v.at[b], acc.at[dst_v.at[j]], add=True)
            nj = j + _NBUF

            @pl.when(nj < _NCHUNK)
            def _():
                pltpu.async_copy(
                    s_hbm.at[src_v.at[pl.ds(nj * _CHUNK, _CHUNK)]],
                    rows_v.at[b], sems[b])
        return carry

    lax.fori_loop(0, _NCHUNK // _NBUF, outer, 0)
    plsc.subcore_barrier()
    pltpu.sync_copy(acc.at[pl.ds(s * _RPT, _RPT)],
                    out_hbm.at[c, pl.ds(s * _RPT, _RPT)])


@functools.cache
def _prop_kernel():
    return pl.kernel(
        _prop_body,
        out_type=jax.ShapeDtypeStruct((_NC, _NPAD, _H), jnp.float32),
        mesh=plsc.VectorSubcoreMesh(core_axis_name="c", subcore_axis_name="s",
                                    num_cores=_NC, num_subcores=_NS),
        scratch_types=[
            pltpu.VMEM((_EPT,), jnp.int32),
            pltpu.VMEM((_NCHUNK, _CHUNK), jnp.int32),
            pltpu.VMEM((_NBUF, _CHUNK, _H), jnp.float32),
            pltpu.VMEM_SHARED((_NPAD, _H), jnp.float32),
            pltpu.SemaphoreType.DMA,
            pltpu.SemaphoreType.DMA,
        ],
    )


def _head_body(x_ref, w_ref, degp_ref, s_ref, dinv_ref):
    deg = degp_ref[0][:, :16] + degp_ref[1][:, :16] + 1.0
    dinv = lax.rsqrt(deg)
    dinv_ref[...] = dinv
    s_ref[...] = jnp.dot(x_ref[...], w_ref[...],
                         preferred_element_type=jnp.float32,
                         precision=lax.Precision.HIGHEST) * dinv[:, :1]


def _mid_body(p_ref, s_ref, dinv_ref, b_ref, w_ref, out_ref):
    dinv = dinv_ref[:, :1]
    h = jnp.maximum(dinv * (p_ref[0] + p_ref[1] + s_ref[...]) + b_ref[...],
                    0.0)
    out_ref[...] = jnp.dot(h, w_ref[...],
                           preferred_element_type=jnp.float32,
                         precision=lax.Precision.HIGHEST) * dinv


def _pool_body(p_ref, s_ref, dinv_ref, b_ref, batch_ref, wf_ref, bf_ref,
               out_ref, pooled_acc, counts_acc):
    i = pl.program_id(0)
    dinv = dinv_ref[:, :1]
    h = jnp.maximum(dinv * (p_ref[0] + p_ref[1] + s_ref[...]) + b_ref[...],
                    0.0)
    bt = batch_ref[0]
    gids = lax.broadcasted_iota(jnp.int32, (_G, 128), 0)
    onehot = (gids == bt).astype(jnp.float32)
    contrib = jnp.dot(onehot, h, preferred_element_type=jnp.float32,
                         precision=lax.Precision.HIGHEST)
    cnt = jnp.sum(onehot, axis=1, keepdims=True)

    @pl.when(i == 0)
    def _():
        pooled_acc[...] = jnp.zeros_like(pooled_acc)
        counts_acc[...] = jnp.zeros_like(counts_acc)

    pooled_acc[...] += contrib
    counts_acc[...] += jnp.broadcast_to(cnt, counts_acc.shape)

    @pl.when(i == _NBLK - 1)
    def _():
        pooled = pooled_acc[...] / jnp.maximum(counts_acc[...], 1.0)
        out_ref[...] = jnp.dot(pooled, wf_ref[...],
                               preferred_element_type=jnp.float32,
                         precision=lax.Precision.HIGHEST) + bf_ref[...]


def _row_spec(h):
    return pl.BlockSpec((128, h), lambda i: (i, 0))


_P_SPEC = pl.BlockSpec((_NC, 128, _H), lambda i: (0, i, 0))
_DINV_SPEC = pl.BlockSpec((128, 16), lambda i: (i, 0))
_FULL_W = pl.BlockSpec((_H, _H), lambda i: (0, 0))
_FULL_B = pl.BlockSpec((1, _H), lambda i: (0, 0))

_head_kernel = pl.pallas_call(
    _head_body,
    grid=(_NBLK,),
    in_specs=[_row_spec(_D), _FULL_W, _P_SPEC],
    out_specs=[_row_spec(_H), _DINV_SPEC],
    out_shape=[jax.ShapeDtypeStruct((_NPAD, _H), jnp.float32),
               jax.ShapeDtypeStruct((_NPAD, 16), jnp.float32)],
)

_mid_kernel = pl.pallas_call(
    _mid_body,
    grid=(_NBLK,),
    in_specs=[_P_SPEC, _row_spec(_H), _DINV_SPEC, _FULL_B, _FULL_W],
    out_specs=_row_spec(_H),
    out_shape=jax.ShapeDtypeStruct((_NPAD, _H), jnp.float32),
)

_pool_kernel = pl.pallas_call(
    _pool_body,
    grid=(_NBLK,),
    in_specs=[_P_SPEC, _row_spec(_H), _DINV_SPEC, _FULL_B,
              pl.BlockSpec((1, 1, 128), lambda i: (i, 0, 0)),
              pl.BlockSpec((_H, _OUT), lambda i: (0, 0)),
              pl.BlockSpec((1, _OUT), lambda i: (0, 0))],
    out_specs=pl.BlockSpec((_G, _OUT), lambda i: (0, 0)),
    out_shape=jax.ShapeDtypeStruct((_G, _OUT), jnp.float32),
    scratch_shapes=[pltpu.VMEM((_G, _H), jnp.float32),
                    pltpu.VMEM((_G, _H), jnp.float32)],
)


@jax.jit
def kernel(x, edge_index, edge_attr, batch, W0, b0, W1, b1, W2, b2, Wf, bf):
    del edge_attr

    src = jnp.concatenate(
        [edge_index[0], jnp.zeros((_EPAD - _E,), jnp.int32)])
    dst = jnp.concatenate(
        [edge_index[1], jnp.full((_EPAD - _E,), _DUMP, jnp.int32)])
    dst3 = dst.reshape(_NW, _NCHUNK, _CHUNK)
    x_pad = jnp.zeros((_NPAD, _D), jnp.float32).at[:_N].set(x)
    batch2 = jnp.full((_NPAD,), -1, jnp.int32).at[:_N].set(batch)
    batch2 = batch2.reshape(_NBLK, 1, 128)
    zeros_nh = jnp.zeros((_NPAD, _H), jnp.float32)
    ones_ch = jnp.ones((_CHUNK, _H), jnp.float32)
    b0r = b0.reshape(1, _H)
    b1r = b1.reshape(1, _H)
    b2r = b2.reshape(1, _H)
    bfr = bf.reshape(1, _OUT)

    degp = _deg_kernel()(dst3, ones_ch, zeros_nh)
    s0, dinv8 = _head_kernel(x_pad, W0, degp)

    prop = _prop_kernel()
    p1 = prop(src, dst3, s0, zeros_nh)
    s1 = _mid_kernel(p1, s0, dinv8, b0r, W1)
    p2 = prop(src, dst3, s1, zeros_nh)
    s2 = _mid_kernel(p2, s1, dinv8, b1r, W2)
    p3 = prop(src, dst3, s2, zeros_nh)

    return _pool_kernel(p3, s2, dinv8, b2r, batch2, Wf, bfr)


# speedup vs baseline: 6.8840x; 1.0004x over previous
import functools
import jax
import jax.numpy as jnp
from jax import lax
from jax.experimental import pallas as pl
from jax.experimental.pallas import tpu as pltpu
from jax.experimental.pallas import tpu_sc as plsc

_N = 10000
_E = 320000
_D = 128
_H = 128
_OUT = 16
_G = 64

_NC = 2
_NS = 16
_NW = _NC * _NS
_CHUNK = 64
_NBUF = 2
_NCHUNK = 160
_EPT = _NCHUNK * _CHUNK
_EPAD = _NW * _EPT
_NPAD = 10112
_NBLK = _NPAD // 128
_RPT = _NPAD // _NS
_DUMP = _NPAD - 1


def _deg_body(dst3_hbm, ones_hbm, zeros8_hbm, degp_hbm, dst_v, ones_v, acc8):
    c = lax.axis_index("c")
    s = lax.axis_index("s")
    tid = c * _NS + s
    pltpu.sync_copy(dst3_hbm.at[tid], dst_v)
    pltpu.sync_copy(ones_hbm, ones_v)
    pltpu.sync_copy(zeros8_hbm.at[pl.ds(s * _RPT, _RPT)],
                    acc8.at[pl.ds(s * _RPT, _RPT)])
    plsc.subcore_barrier()

    def body(j, carry):
        pltpu.sync_copy(ones_v, acc8.at[dst_v.at[j]], add=True)
        return carry

    lax.fori_loop(0, _NCHUNK, body, 0)
    plsc.subcore_barrier()
    pltpu.sync_copy(acc8.at[pl.ds(s * _RPT, _RPT)],
                    degp_hbm.at[c, pl.ds(s * _RPT, _RPT)])


@functools.cache
def _deg_kernel():
    return pl.kernel(
        _deg_body,
        out_type=jax.ShapeDtypeStruct((_NC, _NPAD, _H), jnp.float32),
        mesh=plsc.VectorSubcoreMesh(core_axis_name="c", subcore_axis_name="s",
                                    num_cores=_NC, num_subcores=_NS),
        scratch_types=[
            pltpu.VMEM((_NCHUNK, _CHUNK), jnp.int32),
            pltpu.VMEM((_CHUNK, _H), jnp.float32),
            pltpu.VMEM_SHARED((_NPAD, _H), jnp.float32),
        ],
    )


def _prop_body(src_hbm, dst3_hbm, s_hbm, zeros_hbm, out_hbm,
               src_v, dst_v, rows_v, acc, gs0, gs1, ss0, ss1):
    c = lax.axis_index("c")
    s = lax.axis_index("s")
    tid = c * _NS + s
    gs = (gs0, gs1)
    ss = (ss0, ss1)

    def fire_gather(j, b):
        pltpu.async_copy(s_hbm.at[src_v.at[pl.ds(j * _CHUNK, _CHUNK)]],
                         rows_v.at[b], gs[b])

    def wait_gather(j, b):
        pltpu.make_async_copy(s_hbm.at[src_v.at[pl.ds(j * _CHUNK, _CHUNK)]],
                              rows_v.at[b], gs[b]).wait()

    def fire_scatter(j, b):
        pltpu.async_copy(rows_v.at[b], acc.at[dst_v.at[j]], ss[b], add=True)

    def wait_scatter(b):
        pltpu.make_async_copy(rows_v.at[b], acc.at[dst_v.at[0]], ss[b]).wait()

    pltpu.sync_copy(src_hbm.at[pl.ds(tid * _EPT, _EPT)], src_v)
    pltpu.sync_copy(dst3_hbm.at[tid], dst_v)
    for b in range(_NBUF):
        fire_gather(b, b)
    pltpu.sync_copy(zeros_hbm.at[pl.ds(s * _RPT, _RPT)],
                    acc.at[pl.ds(s * _RPT, _RPT)])
    plsc.subcore_barrier()

    def step(j, b):
        nxt = j + 1
        if not (isinstance(nxt, int) and nxt >= _NCHUNK):

            @pl.when(jnp.logical_and(nxt >= _NBUF, nxt < _NCHUNK))
            def _():
                wait_scatter((b + 1) % _NBUF)
                fire_gather(nxt, (b + 1) % _NBUF)

        wait_gather(j, b)
        fire_scatter(j, b)

    n_main = (_NCHUNK // _NBUF) * _NBUF

    def outer(jo, carry):
        for b in range(_NBUF):
            step(jo * _NBUF + b, b)
        return carry

    lax.fori_loop(0, n_main // _NBUF, outer, 0)
    for j in range(n_main, _NCHUNK):
        step(j, j % _NBUF)
    for b in range(_NBUF):
        wait_scatter(b)
    plsc.subcore_barrier()
    pltpu.sync_copy(acc.at[pl.ds(s * _RPT, _RPT)],
                    out_hbm.at[c, pl.ds(s * _RPT, _RPT)])


@functools.cache
def _prop_kernel():
    return pl.kernel(
        _prop_body,
        out_type=jax.ShapeDtypeStruct((_NC, _NPAD, _H), jnp.float32),
        mesh=plsc.VectorSubcoreMesh(core_axis_name="c", subcore_axis_name="s",
                                    num_cores=_NC, num_subcores=_NS),
        scratch_types=[
            pltpu.VMEM((_EPT,), jnp.int32),
            pltpu.VMEM((_NCHUNK, _CHUNK), jnp.int32),
            pltpu.VMEM((_NBUF, _CHUNK, _H), jnp.float32),
            pltpu.VMEM_SHARED((_NPAD, _H), jnp.float32),
        ] + [pltpu.SemaphoreType.DMA] * (2 * _NBUF),
    )


def _head_body(x_ref, w_ref, degp_ref, s_ref, dinv_ref):
    deg = degp_ref[0][:, :16] + degp_ref[1][:, :16] + 1.0
    dinv = lax.rsqrt(deg)
    dinv_ref[...] = dinv
    s_ref[...] = jnp.dot(x_ref[...], w_ref[...],
                         preferred_element_type=jnp.float32,
                         precision=lax.Precision.HIGHEST) * dinv[:, :1]


def _mid_body(p_ref, s_ref, dinv_ref, b_ref, w_ref, out_ref):
    dinv = dinv_ref[:, :1]
    h = jnp.maximum(dinv * (p_ref[0] + p_ref[1] + s_ref[...]) + b_ref[...],
                    0.0)
    out_ref[...] = jnp.dot(h, w_ref[...],
                           preferred_element_type=jnp.float32,
                         precision=lax.Precision.HIGHEST) * dinv


def _pool_body(p_ref, s_ref, dinv_ref, b_ref, batch_ref, wf_ref, bf_ref,
               out_ref, pooled_acc, counts_acc):
    i = pl.program_id(0)
    dinv = dinv_ref[:, :1]
    h = jnp.maximum(dinv * (p_ref[0] + p_ref[1] + s_ref[...]) + b_ref[...],
                    0.0)
    bt = batch_ref[0]
    gids = lax.broadcasted_iota(jnp.int32, (_G, 128), 0)
    onehot = (gids == bt).astype(jnp.float32)
    contrib = jnp.dot(onehot, h, preferred_element_type=jnp.float32,
                         precision=lax.Precision.HIGHEST)
    cnt = jnp.sum(onehot, axis=1, keepdims=True)

    @pl.when(i == 0)
    def _():
        pooled_acc[...] = jnp.zeros_like(pooled_acc)
        counts_acc[...] = jnp.zeros_like(counts_acc)

    pooled_acc[...] += contrib
    counts_acc[...] += jnp.broadcast_to(cnt, counts_acc.shape)

    @pl.when(i == _NBLK - 1)
    def _():
        pooled = pooled_acc[...] / jnp.maximum(counts_acc[...], 1.0)
        out_ref[...] = jnp.dot(pooled, wf_ref[...],
                               preferred_element_type=jnp.float32,
                         precision=lax.Precision.HIGHEST) + bf_ref[...]


def _row_spec(h):
    return pl.BlockSpec((128, h), lambda i: (i, 0))


_P_SPEC = pl.BlockSpec((_NC, 128, _H), lambda i: (0, i, 0))
_DINV_SPEC = pl.BlockSpec((128, 16), lambda i: (i, 0))
_FULL_W = pl.BlockSpec((_H, _H), lambda i: (0, 0))
_FULL_B = pl.BlockSpec((1, _H), lambda i: (0, 0))

_head_kernel = pl.pallas_call(
    _head_body,
    grid=(_NBLK,),
    in_specs=[_row_spec(_D), _FULL_W, _P_SPEC],
    out_specs=[_row_spec(_H), _DINV_SPEC],
    out_shape=[jax.ShapeDtypeStruct((_NPAD, _H), jnp.float32),
               jax.ShapeDtypeStruct((_NPAD, 16), jnp.float32)],
)

_mid_kernel = pl.pallas_call(
    _mid_body,
    grid=(_NBLK,),
    in_specs=[_P_SPEC, _row_spec(_H), _DINV_SPEC, _FULL_B, _FULL_W],
    out_specs=_row_spec(_H),
    out_shape=jax.ShapeDtypeStruct((_NPAD, _H), jnp.float32),
)

_pool_kernel = pl.pallas_call(
    _pool_body,
    grid=(_NBLK,),
    in_specs=[_P_SPEC, _row_spec(_H), _DINV_SPEC, _FULL_B,
              pl.BlockSpec((1, 1, 128), lambda i: (i, 0, 0)),
              pl.BlockSpec((_H, _OUT), lambda i: (0, 0)),
              pl.BlockSpec((1, _OUT), lambda i: (0, 0))],
    out_specs=pl.BlockSpec((_G, _OUT), lambda i: (0, 0)),
    out_shape=jax.ShapeDtypeStruct((_G, _OUT), jnp.float32),
    scratch_shapes=[pltpu.VMEM((_G, _H), jnp.float32),
                    pltpu.VMEM((_G, _H), jnp.float32)],
)


@jax.jit
def kernel(x, edge_index, edge_attr, batch, W0, b0, W1, b1, W2, b2, Wf, bf):
    del edge_attr

    src = jnp.concatenate(
        [edge_index[0], jnp.zeros((_EPAD - _E,), jnp.int32)])
    dst = jnp.concatenate(
        [edge_index[1], jnp.full((_EPAD - _E,), _DUMP, jnp.int32)])
    dst3 = dst.reshape(_NW, _NCHUNK, _CHUNK)
    x_pad = jnp.zeros((_NPAD, _D), jnp.float32).at[:_N].set(x)
    batch2 = jnp.full((_NPAD,), -1, jnp.int32).at[:_N].set(batch)
    batch2 = batch2.reshape(_NBLK, 1, 128)
    zeros_nh = jnp.zeros((_NPAD, _H), jnp.float32)
    ones_ch = jnp.ones((_CHUNK, _H), jnp.float32)
    b0r = b0.reshape(1, _H)
    b1r = b1.reshape(1, _H)
    b2r = b2.reshape(1, _H)
    bfr = bf.reshape(1, _OUT)

    degp = _deg_kernel()(dst3, ones_ch, zeros_nh)
    s0, dinv8 = _head_kernel(x_pad, W0, degp)

    prop = _prop_kernel()
    p1 = prop(src, dst3, s0, zeros_nh)
    s1 = _mid_kernel(p1, s0, dinv8, b0r, W1)
    p2 = prop(src, dst3, s1, zeros_nh)
    s2 = _mid_kernel(p2, s1, dinv8, b1r, W2)
    p3 = prop(src, dst3, s2, zeros_nh)

    return _pool_kernel(p3, s2, dinv8, b2r, batch2, Wf, bfr)
